# baseline (speedup 1.0000x reference)
import functools

import jax
import jax.numpy as jnp
from jax import lax
from jax.experimental import pallas as pl
from jax.experimental.pallas import tpu as pltpu
from jax.experimental.pallas import tpu_sc as plsc

_N = 10000
_E = 160000
_DF = 256
_DH = 512

_NC = 2
_NS = 16
_LN = 16
_CB = 128

_NP = 10240
_EP = 163840
_ND = 10496
_TRASH = _NP

_C = 1024
_K = 5
_G = 32
_RND = 1024
_LB = _RND + 2 * _G
_ET = _EP // _NS
_NRND = _ET // _RND
_R = _C // _NS
_NROW = _NC * _NS * _K * _NRND


def _mesh():
    return plsc.VectorSubcoreMesh(
        core_axis_name="c", subcore_axis_name="s",
        num_cores=_NC, num_subcores=_NS)


_PARAMS = pltpu.CompilerParams(needs_layout_passes=False)


_DEG_EPT = _EP // (_NC * _NS)
_DEG_G = 128
_DEG_RND = 1024
_DEG_ZR = _ND // _NS


def _deg_body(dst_hbm, degp_hbm, dst_v, idx_v, ones_v, zbuf_v, deg_sh):
    c = lax.axis_index("c")
    s = lax.axis_index("s")
    wid = c * _NS + s
    zero = jnp.zeros((_LN,), jnp.float32)
    for j in range(_DEG_ZR // _LN):
        zbuf_v[pl.ds(j * _LN, _LN)] = zero
    one = jnp.ones((_LN,), jnp.float32)
    for j in range(_DEG_G // _LN):
        ones_v[pl.ds(j * _LN, _LN)] = one
    pltpu.sync_copy(zbuf_v, deg_sh.at[pl.ds(s * _DEG_ZR, _DEG_ZR)])
    plsc.subcore_barrier()

    for r in range(_DEG_EPT // _DEG_RND):
        pltpu.sync_copy(
            dst_hbm.at[pl.ds(wid * _DEG_EPT + r * _DEG_RND, _DEG_RND)], dst_v)

        def body(g, carry):
            for j in range(_DEG_G // _LN):
                idx_v[pl.ds(j * _LN, _LN)] = \
                    dst_v[pl.ds(g * _DEG_G + j * _LN, _LN)]
            pltpu.sync_copy(ones_v, deg_sh.at[idx_v], add=True)
            return carry

        lax.fori_loop(0, _DEG_RND // _DEG_G, body, 0)
    plsc.subcore_barrier()
    pltpu.sync_copy(deg_sh.at[pl.ds(s * _DEG_ZR, _DEG_ZR)], zbuf_v)
    pltpu.sync_copy(zbuf_v, degp_hbm.at[pl.ds(c * _ND + s * _DEG_ZR, _DEG_ZR)])


_deg_call = functools.partial(
    pl.kernel,
    out_type=jax.ShapeDtypeStruct((_NC * _ND,), jnp.float32),
    mesh=_mesh(),
    compiler_params=_PARAMS,
    scratch_types=[
        pltpu.VMEM((_DEG_RND,), jnp.int32),
        pltpu.VMEM((_DEG_G,), jnp.int32),
        pltpu.VMEM((_DEG_G,), jnp.float32),
        pltpu.VMEM((_DEG_ZR,), jnp.float32),
        pltpu.VMEM_SHARED((_ND,), jnp.float32),
    ],
)(_deg_body)


def _process_groups(NB, hs, accs, stages, sidxs, didxs, gsems, ssems,
                    lsrc_v, ldst_v, n):
    ng2 = jnp.maximum(1, (n + 2 * _G - 1) // (2 * _G))

    def load_idx(par, g):
        for j in range(_G // _LN):
            sidxs[par][pl.ds(j * _LN, _LN)] = \
                lsrc_v[pl.ds(g * _G + j * _LN, _LN)]
            didxs[par][pl.ds(j * _LN, _LN)] = \
                ldst_v[pl.ds(g * _G + j * _LN, _LN)]

    def fire_g(par):
        for b in range(NB):
            pltpu.async_copy(hs[b].at[sidxs[par]], stages[par][b], gsems[par])

    def wait_g(par):
        for b in range(NB):
            pltpu.make_async_copy(
                hs[b].at[sidxs[par]], stages[par][b], gsems[par]).wait()

    def fire_s(par):
        for b in range(NB):
            pltpu.async_copy(stages[par][b], accs[b].at[didxs[par]],
                             ssems[par], add=True)

    def wait_s(par):
        for b in range(NB):
            pltpu.make_async_copy(
                stages[par][b], accs[b].at[didxs[par]], ssems[par]).wait()

    load_idx(0, 0)
    fire_g(0)
    load_idx(1, 1)
    fire_g(1)
    wait_g(0)
    fire_s(0)
    wait_g(1)
    fire_s(1)

    def pbody(p, carry):
        wait_s(0)
        load_idx(0, 2 * p)
        fire_g(0)
        wait_s(1)
        load_idx(1, 2 * p + 1)
        fire_g(1)
        wait_g(0)
        fire_s(0)
        wait_g(1)
        fire_s(1)
        return carry

    lax.fori_loop(1, ng2, pbody, 0)
    wait_s(0)
    wait_s(1)


def _agg1_body(src_hbm, dst_hbm, h0, h1, h2, h3,
               o0, o1, o2, o3, lsrc_hbm, ldst_hbm, cnt_hbm,
               ebs0, ebd0, ebs1, ebd1, lsrc_v, ldst_v, cwrite,
               sidx0, didx0, sidx1, didx1,
               st00, st01, st02, st03, st10, st11, st12, st13,
               ac0, ac1, ac2, ac3,
               esem0, esem1, gsem0, gsem1, ssem0, ssem1):
    NB = 4
    hs = (h0, h1, h2, h3)
    outs = (o0, o1, o2, o3)
    ebs = (ebs0, ebs1)
    ebd = (ebd0, ebd1)
    stages = ((st00, st01, st02, st03), (st10, st11, st12, st13))
    accs = (ac0, ac1, ac2, ac3)
    sidxs = (sidx0, sidx1)
    didxs = (didx0, didx1)
    esems = (esem0, esem1)
    gsems = (gsem0, gsem1)
    ssems = (ssem0, ssem1)

    c = lax.axis_index("c")
    s = lax.axis_index("s")
    base = c * (_K * _C)
    r0 = s * _R
    iota = jnp.arange(_LN, dtype=jnp.int32)

    def fire_edges(r, par):
        e0 = s * _ET + r * _RND
        pltpu.async_copy(src_hbm.at[pl.ds(e0, _RND)], ebs[par], esems[par])
        pltpu.async_copy(dst_hbm.at[pl.ds(e0, _RND)], ebd[par], esems[par])

    def wait_edges(r, par):
        e0 = s * _ET + r * _RND
        pltpu.make_async_copy(src_hbm.at[pl.ds(e0, _RND)], ebs[par],
                              esems[par]).wait()
        pltpu.make_async_copy(dst_hbm.at[pl.ds(e0, _RND)], ebd[par],
                              esems[par]).wait()

    def do_round(k, lo, r, par):
        def cbody(i, cnt):
            sv = ebs[par][pl.ds(i * _LN, _LN)]
            dv = ebd[par][pl.ds(i * _LN, _LN)]
            m = (dv >= lo) & (dv < lo + _C)
            inc = plsc.cumsum(jnp.where(m, 1, 0).astype(jnp.int32))
            pos = cnt + inc - 1
            plsc.store_scatter(lsrc_v, [pos], sv, mask=m)
            plsc.store_scatter(ldst_v, [pos], dv - lo, mask=m)
            return cnt + plsc.all_reduce_population_count(m)

        cnt = lax.fori_loop(0, _RND // _LN, cbody,
                            jnp.zeros((_LN,), jnp.int32))
        for j in range(2 * _G // _LN):
            tail = cnt + iota + j * _LN
            plsc.store_scatter(lsrc_v, [tail], jnp.zeros((_LN,), jnp.int32))
            plsc.store_scatter(ldst_v, [tail], jnp.full((_LN,), _C, jnp.int32))

        row = ((c * _NS + s) * _K + k) * _NRND + r
        pltpu.sync_copy(lsrc_v, lsrc_hbm.at[pl.ds(row * _LB, _LB)])
        pltpu.sync_copy(ldst_v, ldst_hbm.at[pl.ds(row * _LB, _LB)])
        cwrite[pl.ds(0, _LN)] = cnt
        pltpu.sync_copy(cwrite, cnt_hbm.at[pl.ds(row * _LN, _LN)])

        _process_groups(NB, hs, accs, stages, sidxs, didxs, gsems, ssems,
                        lsrc_v, ldst_v, cnt[0])

    def phase(k, carry):
        lo = base + k * _C
        for b in range(NB):
            pltpu.sync_copy(hs[b].at[pl.ds(lo + r0, _R)],
                            accs[b].at[pl.ds(r0, _R)])
        plsc.subcore_barrier()

        fire_edges(0, 0)

        def rpair(p, carry2):
            wait_edges(2 * p, 0)
            fire_edges(2 * p + 1, 1)
            do_round(k, lo, 2 * p, 0)
            wait_edges(2 * p + 1, 1)
            fire_edges(2 * p + 2, 0)
            do_round(k, lo, 2 * p + 1, 1)
            return carry2

        lax.fori_loop(0, _NRND // 2, rpair, 0)
        wait_edges(_NRND, 0)
        plsc.subcore_barrier()
        for b in range(NB):
            pltpu.sync_copy(accs[b].at[pl.ds(r0, _R)],
                            outs[b].at[pl.ds(lo + r0, _R)])
        plsc.subcore_barrier()
        return carry

    lax.fori_loop(0, _K, phase, 0)


_agg1_call = functools.partial(
    pl.kernel,
    out_type=tuple([jax.ShapeDtypeStruct((_NP, _CB), jnp.float32)] * 4
                   + [jax.ShapeDtypeStruct(((_NROW + 1) * _LB,), jnp.int32),
                      jax.ShapeDtypeStruct(((_NROW + 1) * _LB,), jnp.int32),
                      jax.ShapeDtypeStruct((_NROW * _LN,), jnp.int32)]),
    mesh=_mesh(),
    compiler_params=_PARAMS,
    scratch_types=(
        [pltpu.VMEM((_RND,), jnp.int32)] * 4
        + [pltpu.VMEM((_LB,), jnp.int32)] * 2
        + [pltpu.VMEM((_LN,), jnp.int32)]
        + [pltpu.VMEM((_G,), jnp.int32)] * 4
        + [pltpu.VMEM((_G, _CB), jnp.float32)] * 8
        + [pltpu.VMEM_SHARED((_C + 8, _CB), jnp.float32)] * 4
        + [pltpu.SemaphoreType.DMA] * 6
    ),
)(_agg1_body)


def _agg2_body(lsrc_hbm, ldst_hbm, cnt_hbm, h0, h1,
               o0, o1,
               ls0, ld0, ls1, ld1, cbuf,
               sidx0, didx0, sidx1, didx1,
               st00, st01, st10, st11,
               ac0, ac1,
               lsem0, lsem1, gsem0, gsem1, ssem0, ssem1):
    NB = 2
    hs = (h0, h1)
    outs = (o0, o1)
    lsv = (ls0, ls1)
    ldv = (ld0, ld1)
    stages = ((st00, st01), (st10, st11))
    accs = (ac0, ac1)
    sidxs = (sidx0, sidx1)
    didxs = (didx0, didx1)
    lsems = (lsem0, lsem1)
    gsems = (gsem0, gsem1)
    ssems = (ssem0, ssem1)

    c = lax.axis_index("c")
    s = lax.axis_index("s")
    base = c * (_K * _C)
    r0 = s * _R

    pltpu.sync_copy(
        cnt_hbm.at[pl.ds((c * _NS + s) * _K * _NRND * _LN,
                         _K * _NRND * _LN)], cbuf)

    def fire_lists(k, r, par):
        row = ((c * _NS + s) * _K + k) * _NRND + r
        pltpu.async_copy(lsrc_hbm.at[pl.ds(row * _LB, _LB)], lsv[par],
                         lsems[par])
        pltpu.async_copy(ldst_hbm.at[pl.ds(row * _LB, _LB)], ldv[par],
                         lsems[par])

    def wait_lists(k, r, par):
        row = ((c * _NS + s) * _K + k) * _NRND + r
        pltpu.make_async_copy(lsrc_hbm.at[pl.ds(row * _LB, _LB)], lsv[par],
                              lsems[par]).wait()
        pltpu.make_async_copy(ldst_hbm.at[pl.ds(row * _LB, _LB)], ldv[par],
                              lsems[par]).wait()

    def do_round(k, r, par):
        n = cbuf[pl.ds((k * _NRND + r) * _LN, _LN)][0]
        _process_groups(NB, hs, accs, stages, sidxs, didxs, gsems, ssems,
                        lsv[par], ldv[par], n)

    def phase(k, carry):
        lo = base + k * _C
        for b in range(NB):
            pltpu.sync_copy(hs[b].at[pl.ds(lo + r0, _R)],
                            accs[b].at[pl.ds(r0, _R)])
        plsc.subcore_barrier()

        fire_lists(k, 0, 0)

        def rpair(p, carry2):
            wait_lists(k, 2 * p, 0)
            fire_lists(k, 2 * p + 1, 1)
            do_round(k, 2 * p, 0)
            wait_lists(k, 2 * p + 1, 1)
            fire_lists(k, 2 * p + 2, 0)
            do_round(k, 2 * p + 1, 1)
            return carry2

        lax.fori_loop(0, _NRND // 2, rpair, 0)
        wait_lists(k, _NRND, 0)
        plsc.subcore_barrier()
        for b in range(NB):
            pltpu.sync_copy(accs[b].at[pl.ds(r0, _R)],
                            outs[b].at[pl.ds(lo + r0, _R)])
        plsc.subcore_barrier()
        return carry

    lax.fori_loop(0, _K, phase, 0)


_agg2_call = functools.partial(
    pl.kernel,
    out_type=tuple([jax.ShapeDtypeStruct((_NP, _CB), jnp.float32)] * 2),
    mesh=_mesh(),
    compiler_params=_PARAMS,
    scratch_types=(
        [pltpu.VMEM((_LB,), jnp.int32)] * 4
        + [pltpu.VMEM((_K * _NRND * _LN,), jnp.int32)]
        + [pltpu.VMEM((_G,), jnp.int32)] * 4
        + [pltpu.VMEM((_G, _CB), jnp.float32)] * 4
        + [pltpu.VMEM_SHARED((_C + 8, _CB), jnp.float32)] * 2
        + [pltpu.SemaphoreType.DMA] * 6
    ),
)(_agg2_body)


_BR = 256


def _b1_kernel(x_ref, w_ref, d0_ref, d1_ref, h0, h1, h2, h3, dis_ref):
    deg = d0_ref[...] + d1_ref[...] + 1.0
    dis = lax.rsqrt(deg)
    dis_ref[...] = dis
    h = jnp.dot(x_ref[...] * dis, w_ref[...],
                preferred_element_type=jnp.float32)
    h0[...] = h[:, 0 * _CB:1 * _CB]
    h1[...] = h[:, 1 * _CB:2 * _CB]
    h2[...] = h[:, 2 * _CB:3 * _CB]
    h3[...] = h[:, 3 * _CB:4 * _CB]


def _b1_call(xp, W1, d0, d1):
    blk = pl.BlockSpec((_BR, _CB), lambda i: (i, 0))
    return pl.pallas_call(
        _b1_kernel,
        grid=(_NP // _BR,),
        in_specs=[
            pl.BlockSpec((_BR, _DF), lambda i: (i, 0)),
            pl.BlockSpec((_DF, _DH), lambda i: (0, 0)),
            pl.BlockSpec((_BR, 1), lambda i: (i, 0)),
            pl.BlockSpec((_BR, 1), lambda i: (i, 0)),
        ],
        out_specs=[blk, blk, blk, blk,
                   pl.BlockSpec((_BR, 1), lambda i: (i, 0))],
        out_shape=[jax.ShapeDtypeStruct((_NP, _CB), jnp.float32)] * 4
        + [jax.ShapeDtypeStruct((_NP, 1), jnp.float32)],
    )(xp, W1, d0, d1)


def _b2_kernel(a0, a1, a2, a3, dis_ref, b_ref, w_ref, o0, o1):
    dis = dis_ref[...]
    zs = []
    for b, a in enumerate((a0, a1, a2, a3)):
        t = dis * a[...] + b_ref[:, b * _CB:(b + 1) * _CB]
        zs.append(jnp.maximum(t, 0.0) * dis)
    z = jnp.concatenate(zs, axis=1)
    o = jnp.dot(z, w_ref[...], preferred_element_type=jnp.float32)
    o0[...] = o[:, 0 * _CB:1 * _CB]
    o1[...] = o[:, 1 * _CB:2 * _CB]


def _b2_call(agg1, dis, b1r, W2):
    blk = pl.BlockSpec((_BR, _CB), lambda i: (i, 0))
    return pl.pallas_call(
        _b2_kernel,
        grid=(_NP // _BR,),
        in_specs=[blk] * 4 + [
            pl.BlockSpec((_BR, 1), lambda i: (i, 0)),
            pl.BlockSpec((1, _DH), lambda i: (0, 0)),
            pl.BlockSpec((_DH, _DF), lambda i: (0, 0)),
        ],
        out_specs=[blk, blk],
        out_shape=[jax.ShapeDtypeStruct((_NP, _CB), jnp.float32)] * 2,
    )(*agg1, dis, b1r, W2)


def _b3_kernel(a0, a1, dis_ref, b_ref, o_ref):
    dis = dis_ref[...]
    o_ref[...] = jnp.concatenate(
        [dis * a0[...], dis * a1[...]], axis=1) + b_ref[...]


def _b3_call(agg2, dis, b2r):
    blk = pl.BlockSpec((_BR, _CB), lambda i: (i, 0))
    return pl.pallas_call(
        _b3_kernel,
        grid=(_NP // _BR,),
        in_specs=[blk] * 2 + [
            pl.BlockSpec((_BR, 1), lambda i: (i, 0)),
            pl.BlockSpec((1, _DF), lambda i: (0, 0)),
        ],
        out_specs=pl.BlockSpec((_BR, _DF), lambda i: (i, 0)),
        out_shape=jax.ShapeDtypeStruct((_NP, _DF), jnp.float32),
    )(*agg2, dis, b2r)


@jax.jit
def kernel(x, edge_index, W1, b1, W2, b2):
    src = edge_index[0].astype(jnp.int32)
    dst = edge_index[1].astype(jnp.int32)
    srcp = jnp.concatenate(
        [src, jnp.zeros((_EP + _RND - _E,), jnp.int32)])
    dstp = jnp.concatenate(
        [dst, jnp.full((_EP + _RND - _E,), _TRASH, jnp.int32)])
    xp = jnp.concatenate([x, jnp.zeros((_NP - _N, _DF), x.dtype)])

    degp = _deg_call(dstp)
    d0 = degp[:_NP, None]
    d1 = degp[_ND:_ND + _NP, None]

    *h1s, dis = _b1_call(xp, W1, d0, d1)
    *agg1, lsrc, ldst, cnts = _agg1_call(srcp, dstp, *h1s)
    h2s = _b2_call(agg1, dis, b1[None, :], W2)
    agg2 = _agg2_call(lsrc, ldst, cnts, *h2s)
    out = _b3_call(agg2, dis, b2[None, :])
    return out[:_N]


# speedup vs baseline: 1.4922x; 1.4922x over previous
import functools

import jax
import jax.numpy as jnp
from jax import lax
from jax.experimental import pallas as pl
from jax.experimental.pallas import tpu as pltpu
from jax.experimental.pallas import tpu_sc as plsc

_N = 10000
_E = 160000
_DF = 256
_DH = 512

_NC = 2
_NS = 16
_LN = 16
_CB = 128

_NP = 10240
_EP = 163840
_ND = 10496
_TRASH = _NP

_C = 1024
_K = 5
_G = 32
_RND = 1024
_LB = _RND + 2 * _G
_ET = _EP // _NS
_NRND = _ET // _RND
_R = _C // _NS
_NROW = _NC * _NS * _K * _NRND


def _mesh():
    return plsc.VectorSubcoreMesh(
        core_axis_name="c", subcore_axis_name="s",
        num_cores=_NC, num_subcores=_NS)


_PARAMS = pltpu.CompilerParams(needs_layout_passes=False)


_DEG_EPT = _EP // (_NC * _NS)
_DEG_G = 128
_DEG_RND = 1024
_DEG_ZR = _ND // _NS


def _deg_body(dst_hbm, degp_hbm, dst_v, idx_v, ones_v, zbuf_v, deg_sh):
    c = lax.axis_index("c")
    s = lax.axis_index("s")
    wid = c * _NS + s
    zero = jnp.zeros((_LN,), jnp.float32)
    for j in range(_DEG_ZR // _LN):
        zbuf_v[pl.ds(j * _LN, _LN)] = zero
    one = jnp.ones((_LN,), jnp.float32)
    for j in range(_DEG_G // _LN):
        ones_v[pl.ds(j * _LN, _LN)] = one
    pltpu.sync_copy(zbuf_v, deg_sh.at[pl.ds(s * _DEG_ZR, _DEG_ZR)])
    plsc.subcore_barrier()

    for r in range(_DEG_EPT // _DEG_RND):
        pltpu.sync_copy(
            dst_hbm.at[pl.ds(wid * _DEG_EPT + r * _DEG_RND, _DEG_RND)], dst_v)

        def body(g, carry):
            for j in range(_DEG_G // _LN):
                idx_v[pl.ds(j * _LN, _LN)] = \
                    dst_v[pl.ds(g * _DEG_G + j * _LN, _LN)]
            pltpu.sync_copy(ones_v, deg_sh.at[idx_v], add=True)
            return carry

        lax.fori_loop(0, _DEG_RND // _DEG_G, body, 0)
    plsc.subcore_barrier()
    pltpu.sync_copy(deg_sh.at[pl.ds(s * _DEG_ZR, _DEG_ZR)], zbuf_v)
    pltpu.sync_copy(zbuf_v, degp_hbm.at[pl.ds(c * _ND + s * _DEG_ZR, _DEG_ZR)])


_deg_call = functools.partial(
    pl.kernel,
    out_type=jax.ShapeDtypeStruct((_NC * _ND,), jnp.float32),
    mesh=_mesh(),
    compiler_params=_PARAMS,
    scratch_types=[
        pltpu.VMEM((_DEG_RND,), jnp.int32),
        pltpu.VMEM((_DEG_G,), jnp.int32),
        pltpu.VMEM((_DEG_G,), jnp.float32),
        pltpu.VMEM((_DEG_ZR,), jnp.float32),
        pltpu.VMEM_SHARED((_ND,), jnp.float32),
    ],
)(_deg_body)


def _process_groups(NB, hs, accs, stages, sidxs, didxs, gsems, ssems,
                    lsrc_v, ldst_v, n):
    ng = (n + _G - 1) // _G

    def gbody(g, carry):
        for j in range(_G // _LN):
            sidxs[0][pl.ds(j * _LN, _LN)] = \
                lsrc_v[pl.ds(g * _G + j * _LN, _LN)]
            didxs[0][pl.ds(j * _LN, _LN)] = \
                ldst_v[pl.ds(g * _G + j * _LN, _LN)]
        for b in range(NB):
            pltpu.async_copy(hs[b].at[sidxs[0]], stages[0][b], gsems[0])
        for b in range(NB):
            pltpu.make_async_copy(
                hs[b].at[sidxs[0]], stages[0][b], gsems[0]).wait()
        for b in range(NB):
            pltpu.sync_copy(stages[0][b], accs[b].at[didxs[0]], add=True)
        return carry

    lax.fori_loop(0, ng, gbody, 0)


def _agg1_body(src_hbm, dst_hbm, h0, h1, h2, h3,
               o0, o1, o2, o3, lsrc_hbm, ldst_hbm, cnt_hbm,
               ebs0, ebd0, ebs1, ebd1, lsrc_v, ldst_v, cwrite,
               sidx0, didx0, sidx1, didx1,
               st00, st01, st02, st03, st10, st11, st12, st13,
               ac0, ac1, ac2, ac3,
               esem0, esem1, gsem0, gsem1, ssem0, ssem1):
    NB = 4
    hs = (h0, h1, h2, h3)
    outs = (o0, o1, o2, o3)
    ebs = (ebs0, ebs1)
    ebd = (ebd0, ebd1)
    stages = ((st00, st01, st02, st03), (st10, st11, st12, st13))
    accs = (ac0, ac1, ac2, ac3)
    sidxs = (sidx0, sidx1)
    didxs = (didx0, didx1)
    esems = (esem0, esem1)
    gsems = (gsem0, gsem1)
    ssems = (ssem0, ssem1)

    c = lax.axis_index("c")
    s = lax.axis_index("s")
    base = c * (_K * _C)
    r0 = s * _R
    iota = jnp.arange(_LN, dtype=jnp.int32)

    def fire_edges(r, par):
        e0 = s * _ET + r * _RND
        pltpu.async_copy(src_hbm.at[pl.ds(e0, _RND)], ebs[par], esems[par])
        pltpu.async_copy(dst_hbm.at[pl.ds(e0, _RND)], ebd[par], esems[par])

    def wait_edges(r, par):
        e0 = s * _ET + r * _RND
        pltpu.make_async_copy(src_hbm.at[pl.ds(e0, _RND)], ebs[par],
                              esems[par]).wait()
        pltpu.make_async_copy(dst_hbm.at[pl.ds(e0, _RND)], ebd[par],
                              esems[par]).wait()

    def do_round(k, lo, r, par):
        def cbody(i, cnt):
            sv = ebs[par][pl.ds(i * _LN, _LN)]
            dv = ebd[par][pl.ds(i * _LN, _LN)]
            m = (dv >= lo) & (dv < lo + _C)
            inc = plsc.cumsum(jnp.where(m, 1, 0).astype(jnp.int32))
            pos = cnt + inc - 1
            plsc.store_scatter(lsrc_v, [pos], sv, mask=m)
            plsc.store_scatter(ldst_v, [pos], dv - lo, mask=m)
            return cnt + plsc.all_reduce_population_count(m)

        cnt = lax.fori_loop(0, _RND // _LN, cbody,
                            jnp.zeros((_LN,), jnp.int32))
        for j in range(2 * _G // _LN):
            tail = cnt + iota + j * _LN
            plsc.store_scatter(lsrc_v, [tail], jnp.zeros((_LN,), jnp.int32))
            plsc.store_scatter(ldst_v, [tail], jnp.full((_LN,), _C, jnp.int32))

        row = ((c * _NS + s) * _K + k) * _NRND + r
        pltpu.sync_copy(lsrc_v, lsrc_hbm.at[pl.ds(row * _LB, _LB)])
        pltpu.sync_copy(ldst_v, ldst_hbm.at[pl.ds(row * _LB, _LB)])
        cwrite[pl.ds(0, _LN)] = cnt
        pltpu.sync_copy(cwrite, cnt_hbm.at[pl.ds(row * _LN, _LN)])

        _process_groups(NB, hs, accs, stages, sidxs, didxs, gsems, ssems,
                        lsrc_v, ldst_v, cnt[0])

    def phase(k, carry):
        lo = base + k * _C
        for b in range(NB):
            pltpu.sync_copy(hs[b].at[pl.ds(lo + r0, _R)],
                            accs[b].at[pl.ds(r0, _R)])
        plsc.subcore_barrier()

        fire_edges(0, 0)

        def rpair(p, carry2):
            wait_edges(2 * p, 0)
            fire_edges(2 * p + 1, 1)
            do_round(k, lo, 2 * p, 0)
            wait_edges(2 * p + 1, 1)
            fire_edges(2 * p + 2, 0)
            do_round(k, lo, 2 * p + 1, 1)
            return carry2

        lax.fori_loop(0, _NRND // 2, rpair, 0)
        wait_edges(_NRND, 0)
        plsc.subcore_barrier()
        for b in range(NB):
            pltpu.sync_copy(accs[b].at[pl.ds(r0, _R)],
                            outs[b].at[pl.ds(lo + r0, _R)])
        plsc.subcore_barrier()
        return carry

    lax.fori_loop(0, _K, phase, 0)


_agg1_call = functools.partial(
    pl.kernel,
    out_type=tuple([jax.ShapeDtypeStruct((_NP, _CB), jnp.float32)] * 4
                   + [jax.ShapeDtypeStruct(((_NROW + 1) * _LB,), jnp.int32),
                      jax.ShapeDtypeStruct(((_NROW + 1) * _LB,), jnp.int32),
                      jax.ShapeDtypeStruct((_NROW * _LN,), jnp.int32)]),
    mesh=_mesh(),
    compiler_params=_PARAMS,
    scratch_types=(
        [pltpu.VMEM((_RND,), jnp.int32)] * 4
        + [pltpu.VMEM((_LB,), jnp.int32)] * 2
        + [pltpu.VMEM((_LN,), jnp.int32)]
        + [pltpu.VMEM((_G,), jnp.int32)] * 4
        + [pltpu.VMEM((_G, _CB), jnp.float32)] * 8
        + [pltpu.VMEM_SHARED((_C + 8, _CB), jnp.float32)] * 4
        + [pltpu.SemaphoreType.DMA] * 6
    ),
)(_agg1_body)


def _agg2_body(lsrc_hbm, ldst_hbm, cnt_hbm, h0, h1,
               o0, o1,
               ls0, ld0, ls1, ld1, cbuf,
               sidx0, didx0, sidx1, didx1,
               st00, st01, st10, st11,
               ac0, ac1,
               lsem0, lsem1, gsem0, gsem1, ssem0, ssem1):
    NB = 2
    hs = (h0, h1)
    outs = (o0, o1)
    lsv = (ls0, ls1)
    ldv = (ld0, ld1)
    stages = ((st00, st01), (st10, st11))
    accs = (ac0, ac1)
    sidxs = (sidx0, sidx1)
    didxs = (didx0, didx1)
    lsems = (lsem0, lsem1)
    gsems = (gsem0, gsem1)
    ssems = (ssem0, ssem1)

    c = lax.axis_index("c")
    s = lax.axis_index("s")
    base = c * (_K * _C)
    r0 = s * _R

    pltpu.sync_copy(
        cnt_hbm.at[pl.ds((c * _NS + s) * _K * _NRND * _LN,
                         _K * _NRND * _LN)], cbuf)

    def fire_lists(k, r, par):
        row = ((c * _NS + s) * _K + k) * _NRND + r
        pltpu.async_copy(lsrc_hbm.at[pl.ds(row * _LB, _LB)], lsv[par],
                         lsems[par])
        pltpu.async_copy(ldst_hbm.at[pl.ds(row * _LB, _LB)], ldv[par],
                         lsems[par])

    def wait_lists(k, r, par):
        row = ((c * _NS + s) * _K + k) * _NRND + r
        pltpu.make_async_copy(lsrc_hbm.at[pl.ds(row * _LB, _LB)], lsv[par],
                              lsems[par]).wait()
        pltpu.make_async_copy(ldst_hbm.at[pl.ds(row * _LB, _LB)], ldv[par],
                              lsems[par]).wait()

    def do_round(k, r, par):
        n = cbuf[pl.ds((k * _NRND + r) * _LN, _LN)][0]
        _process_groups(NB, hs, accs, stages, sidxs, didxs, gsems, ssems,
                        lsv[par], ldv[par], n)

    def phase(k, carry):
        lo = base + k * _C
        for b in range(NB):
            pltpu.sync_copy(hs[b].at[pl.ds(lo + r0, _R)],
                            accs[b].at[pl.ds(r0, _R)])
        plsc.subcore_barrier()

        fire_lists(k, 0, 0)

        def rpair(p, carry2):
            wait_lists(k, 2 * p, 0)
            fire_lists(k, 2 * p + 1, 1)
            do_round(k, 2 * p, 0)
            wait_lists(k, 2 * p + 1, 1)
            fire_lists(k, 2 * p + 2, 0)
            do_round(k, 2 * p + 1, 1)
            return carry2

        lax.fori_loop(0, _NRND // 2, rpair, 0)
        wait_lists(k, _NRND, 0)
        plsc.subcore_barrier()
        for b in range(NB):
            pltpu.sync_copy(accs[b].at[pl.ds(r0, _R)],
                            outs[b].at[pl.ds(lo + r0, _R)])
        plsc.subcore_barrier()
        return carry

    lax.fori_loop(0, _K, phase, 0)


_agg2_call = functools.partial(
    pl.kernel,
    out_type=tuple([jax.ShapeDtypeStruct((_NP, _CB), jnp.float32)] * 2),
    mesh=_mesh(),
    compiler_params=_PARAMS,
    scratch_types=(
        [pltpu.VMEM((_LB,), jnp.int32)] * 4
        + [pltpu.VMEM((_K * _NRND * _LN,), jnp.int32)]
        + [pltpu.VMEM((_G,), jnp.int32)] * 4
        + [pltpu.VMEM((_G, _CB), jnp.float32)] * 4
        + [pltpu.VMEM_SHARED((_C + 8, _CB), jnp.float32)] * 2
        + [pltpu.SemaphoreType.DMA] * 6
    ),
)(_agg2_body)


_BR = 256


def _b1_kernel(x_ref, w_ref, d0_ref, d1_ref, h0, h1, h2, h3, dis_ref):
    deg = d0_ref[...] + d1_ref[...] + 1.0
    dis = lax.rsqrt(deg)
    dis_ref[...] = dis
    h = jnp.dot(x_ref[...] * dis, w_ref[...],
                preferred_element_type=jnp.float32)
    h0[...] = h[:, 0 * _CB:1 * _CB]
    h1[...] = h[:, 1 * _CB:2 * _CB]
    h2[...] = h[:, 2 * _CB:3 * _CB]
    h3[...] = h[:, 3 * _CB:4 * _CB]


def _b1_call(xp, W1, d0, d1):
    blk = pl.BlockSpec((_BR, _CB), lambda i: (i, 0))
    return pl.pallas_call(
        _b1_kernel,
        grid=(_NP // _BR,),
        in_specs=[
            pl.BlockSpec((_BR, _DF), lambda i: (i, 0)),
            pl.BlockSpec((_DF, _DH), lambda i: (0, 0)),
            pl.BlockSpec((_BR, 1), lambda i: (i, 0)),
            pl.BlockSpec((_BR, 1), lambda i: (i, 0)),
        ],
        out_specs=[blk, blk, blk, blk,
                   pl.BlockSpec((_BR, 1), lambda i: (i, 0))],
        out_shape=[jax.ShapeDtypeStruct((_NP, _CB), jnp.float32)] * 4
        + [jax.ShapeDtypeStruct((_NP, 1), jnp.float32)],
    )(xp, W1, d0, d1)


def _b2_kernel(a0, a1, a2, a3, dis_ref, b_ref, w_ref, o0, o1):
    dis = dis_ref[...]
    zs = []
    for b, a in enumerate((a0, a1, a2, a3)):
        t = dis * a[...] + b_ref[:, b * _CB:(b + 1) * _CB]
        zs.append(jnp.maximum(t, 0.0) * dis)
    z = jnp.concatenate(zs, axis=1)
    o = jnp.dot(z, w_ref[...], preferred_element_type=jnp.float32)
    o0[...] = o[:, 0 * _CB:1 * _CB]
    o1[...] = o[:, 1 * _CB:2 * _CB]


def _b2_call(agg1, dis, b1r, W2):
    blk = pl.BlockSpec((_BR, _CB), lambda i: (i, 0))
    return pl.pallas_call(
        _b2_kernel,
        grid=(_NP // _BR,),
        in_specs=[blk] * 4 + [
            pl.BlockSpec((_BR, 1), lambda i: (i, 0)),
            pl.BlockSpec((1, _DH), lambda i: (0, 0)),
            pl.BlockSpec((_DH, _DF), lambda i: (0, 0)),
        ],
        out_specs=[blk, blk],
        out_shape=[jax.ShapeDtypeStruct((_NP, _CB), jnp.float32)] * 2,
    )(*agg1, dis, b1r, W2)


def _b3_kernel(a0, a1, dis_ref, b_ref, o_ref):
    dis = dis_ref[...]
    o_ref[...] = jnp.concatenate(
        [dis * a0[...], dis * a1[...]], axis=1) + b_ref[...]


def _b3_call(agg2, dis, b2r):
    blk = pl.BlockSpec((_BR, _CB), lambda i: (i, 0))
    return pl.pallas_call(
        _b3_kernel,
        grid=(_NP // _BR,),
        in_specs=[blk] * 2 + [
            pl.BlockSpec((_BR, 1), lambda i: (i, 0)),
            pl.BlockSpec((1, _DF), lambda i: (0, 0)),
        ],
        out_specs=pl.BlockSpec((_BR, _DF), lambda i: (i, 0)),
        out_shape=jax.ShapeDtypeStruct((_NP, _DF), jnp.float32),
    )(*agg2, dis, b2r)


@jax.jit
def kernel(x, edge_index, W1, b1, W2, b2):
    src = edge_index[0].astype(jnp.int32)
    dst = edge_index[1].astype(jnp.int32)
    srcp = jnp.concatenate(
        [src, jnp.zeros((_EP + _RND - _E,), jnp.int32)])
    dstp = jnp.concatenate(
        [dst, jnp.full((_EP + _RND - _E,), _TRASH, jnp.int32)])
    xp = jnp.concatenate([x, jnp.zeros((_NP - _N, _DF), x.dtype)])

    degp = _deg_call(dstp)
    d0 = degp[:_NP, None]
    d1 = degp[_ND:_ND + _NP, None]

    *h1s, dis = _b1_call(xp, W1, d0, d1)
    *agg1, lsrc, ldst, cnts = _agg1_call(srcp, dstp, *h1s)
    h2s = _b2_call(agg1, dis, b1[None, :], W2)
    agg2 = _agg2_call(lsrc, ldst, cnts, *h2s)
    out = _b3_call(agg2, dis, b2[None, :])
    return out[:_N]
